# Initial kernel scaffold; baseline (speedup 1.0000x reference)
import jax
import jax.numpy as jnp
from jax.experimental import pallas as pl


def kernel(x, lg_x, edge_index, lg_edge_index, node_Ws, node_bs, node_Wk, node_bk, node_Wf, node_bf, node_W1, node_b1, node_W2, node_b2, node_g1, node_be1, node_g2, node_be2, edge_Ws, edge_bs, edge_Wk, edge_bk, edge_Wf, edge_bf, edge_W1, edge_b1, edge_W2, edge_b2, edge_g1, edge_be1, edge_g2, edge_be2):
    raise NotImplementedError("write your pallas kernel here")


# speedup vs baseline: 1.0157x; 1.0157x over previous
import functools

import jax
import jax.numpy as jnp
from jax import lax
from jax.experimental import pallas as pl
from jax.experimental.pallas import tpu as pltpu

_D = 128
_KH = 4
_C = 0.7978845608028654


def _gelu(v):
    return 0.5 * v * (1.0 + jnp.tanh(_C * (v + 0.044715 * v * v * v)))


def _lnorm(v, g, b):
    m = jnp.mean(v, axis=-1, keepdims=True)
    c = v - m
    var = jnp.mean(c * c, axis=-1, keepdims=True)
    return c * lax.rsqrt(var + 1e-5) * g + b


def _dense_body(x_ref, z0_ref, z1_ref, z2_ref, z3_ref, ea_ref,
                Wc_ref, W1_ref, W2_ref,
                b0_ref, b1_ref, b2_ref, g1_ref, be1_ref, g2_ref, be2_ref,
                o_ref):
    x = x_ref[...]
    h = jnp.concatenate(
        [x, z0_ref[...], z1_ref[...], z2_ref[...], z3_ref[...], ea_ref[...]],
        axis=1)
    acc = jnp.dot(h, Wc_ref[...], preferred_element_type=jnp.float32)
    acc = acc + b0_ref[...]
    out = _gelu(acc)
    h1 = _lnorm(x + out, g1_ref[...], be1_ref[...])
    f = _gelu(jnp.dot(h1, W1_ref[...], preferred_element_type=jnp.float32)
              + b1_ref[...])
    f = jnp.dot(f, W2_ref[...], preferred_element_type=jnp.float32) + b2_ref[...]
    o_ref[...] = _lnorm(h1 + f, g2_ref[...], be2_ref[...])


def _dense_core(x, z0, z1, z2, z3, ea, Wc, W1, W2, b0, b1, b2,
                g1, be1, g2, be2, blk):
    R = x.shape[0]
    Rp = ((R + blk - 1) // blk) * blk
    if Rp != R:
        pad = ((0, Rp - R), (0, 0))
        x = jnp.pad(x, pad)
        z0 = jnp.pad(z0, pad)
        z1 = jnp.pad(z1, pad)
        z2 = jnp.pad(z2, pad)
        z3 = jnp.pad(z3, pad)
        ea = jnp.pad(ea, pad)
    row_spec = pl.BlockSpec((blk, _D), lambda i: (i, 0))
    full = lambda shape: pl.BlockSpec(shape, lambda i: (0,) * len(shape))
    out = pl.pallas_call(
        _dense_body,
        grid=(Rp // blk,),
        in_specs=[
            row_spec, row_spec, row_spec, row_spec, row_spec, row_spec,
            full((6 * _D, _D)), full((_D, 4 * _D)), full((4 * _D, _D)),
            full((1, _D)), full((1, 4 * _D)), full((1, _D)),
            full((1, _D)), full((1, _D)), full((1, _D)), full((1, _D)),
        ],
        out_specs=row_spec,
        out_shape=jax.ShapeDtypeStruct((Rp, _D), jnp.float32),
    )(x, z0, z1, z2, z3, ea, Wc, W1, W2, b0, b1, b2, g1, be1, g2, be2)
    return out[:R] if Rp != R else out


def _half(xx, ea, src, dst, n, Ws, bs, Wk, bk, Wf, bf,
          W1, b1, W2, b2, g1, be1, g2, be2, blk):
    zs = []
    z = xx
    for _ in range(_KH):
        z = jax.ops.segment_sum(z[src], dst, num_segments=n)
        zs.append(z)
    Wc = jnp.concatenate([Ws, Wk[0], Wk[1], Wk[2], Wk[3], Wf], axis=0)
    b0 = (bs + bk.sum(axis=0) + bf)[None, :]
    return _dense_core(xx, zs[0], zs[1], zs[2], zs[3], ea, Wc, W1, W2,
                       b0, b1[None, :], b2[None, :],
                       g1[None, :], be1[None, :], g2[None, :], be2[None, :],
                       blk)


def kernel(x, lg_x, edge_index, lg_edge_index,
           node_Ws, node_bs, node_Wk, node_bk, node_Wf, node_bf,
           node_W1, node_b1, node_W2, node_b2, node_g1, node_be1,
           node_g2, node_be2,
           edge_Ws, edge_bs, edge_Wk, edge_bk, edge_Wf, edge_bf,
           edge_W1, edge_b1, edge_W2, edge_b2, edge_g1, edge_be1,
           edge_g2, edge_be2):
    n = x.shape[0]
    src = edge_index[0]
    dst = edge_index[1]
    lsrc = lg_edge_index[0]
    ldst = lg_edge_index[1]
    node_edge_agg = jnp.zeros_like(x).at[src].add(lg_x).at[dst].add(lg_x)
    edge_node_agg = x[src] + x[dst]
    next_x = _half(x, node_edge_agg, src, dst, n,
                   node_Ws, node_bs, node_Wk, node_bk, node_Wf, node_bf,
                   node_W1, node_b1, node_W2, node_b2,
                   node_g1, node_be1, node_g2, node_be2, blk=512)
    next_lg_x = _half(lg_x, edge_node_agg, lsrc, ldst, lg_x.shape[0],
                      edge_Ws, edge_bs, edge_Wk, edge_bk, edge_Wf, edge_bf,
                      edge_W1, edge_b1, edge_W2, edge_b2,
                      edge_g1, edge_be1, edge_g2, edge_be2, blk=512)
    return (next_x, next_lg_x)


# speedup vs baseline: 1.3922x; 1.3706x over previous
import functools

import jax
import jax.numpy as jnp
from jax import lax
from jax.experimental import pallas as pl
from jax.experimental.pallas import tpu as pltpu
from jax.experimental.pallas import tpu_sc as plsc

_D = 128
_KH = 4
_C = 0.7978845608028654
_NSC = 2
_NT = 16
_B = 128


def _seg_partial(table, gidx, dstv, acc_rows):
    m = gidx.shape[0]
    assert m % (_NSC * _NT * _B) == 0
    assert acc_rows % (_NT * 8) == 0
    mt = m // (_NSC * _NT)
    nb = mt // _B
    rpt = acc_rows // _NT
    zeros = jnp.zeros((acc_rows, _D), jnp.float32)
    mesh = plsc.VectorSubcoreMesh(core_axis_name="c", subcore_axis_name="s")

    def body(table_r, gidx_r, dst_r, zero_r, out_r, gi_v, di_v, rows_v,
             acc, sem):
        c = lax.axis_index("c")
        t = lax.axis_index("s")
        r0 = t * rpt
        pltpu.sync_copy(zero_r.at[pl.ds(r0, rpt)], acc.at[pl.ds(r0, rpt)])
        plsc.subcore_barrier()
        base = c * (m // _NSC) + t * mt

        def step(j, carry):
            g0 = base + j * _B
            pltpu.sync_copy(gidx_r.at[pl.ds(g0, _B)], gi_v)
            pltpu.sync_copy(dst_r.at[pl.ds(g0, _B)], di_v)
            pltpu.async_copy(table_r.at[gi_v], rows_v, sem).wait()
            pltpu.sync_copy(rows_v, acc.at[di_v], add=True)
            return carry

        lax.fori_loop(0, nb, step, 0)
        plsc.subcore_barrier()
        pltpu.sync_copy(acc.at[pl.ds(r0, rpt)],
                        out_r.at[c].at[pl.ds(r0, rpt)])

    fn = pl.kernel(
        body,
        out_type=jax.ShapeDtypeStruct((_NSC, acc_rows, _D), jnp.float32),
        mesh=mesh,
        scratch_types=[
            pltpu.VMEM((_B,), jnp.int32),
            pltpu.VMEM((_B,), jnp.int32),
            pltpu.VMEM((_B, _D), jnp.float32),
            pltpu.VMEM_SHARED((acc_rows + 8, _D), jnp.float32),
            pltpu.SemaphoreType.DMA,
        ],
    )
    return fn(table, gidx, dstv, zeros)


def _merge_body(a_ref, b_ref, o_ref):
    o_ref[...] = a_ref[...] + b_ref[...]


def _merge(p):
    rows = p.shape[1]
    blk = 512
    spec = pl.BlockSpec((blk, _D), lambda i: (i, 0))
    return pl.pallas_call(
        _merge_body,
        grid=(rows // blk,),
        in_specs=[spec, spec],
        out_specs=spec,
        out_shape=jax.ShapeDtypeStruct((rows, _D), jnp.float32),
    )(p[0], p[1])


def _pad_edges(gidx, dstv, trash_row):
    m = gidx.shape[0]
    unit = _NSC * _NT * _B
    mp = ((m + unit - 1) // unit) * unit
    if mp != m:
        npad = mp - m
        gidx = jnp.concatenate([gidx, jnp.arange(npad, dtype=jnp.int32) % 8])
        dstv = jnp.concatenate(
            [dstv, jnp.full((npad,), trash_row, jnp.int32)])
    return gidx, dstv


def _gelu(v):
    return 0.5 * v * (1.0 + jnp.tanh(_C * (v + 0.044715 * v * v * v)))


def _lnorm(v, g, b):
    m = jnp.mean(v, axis=-1, keepdims=True)
    c = v - m
    var = jnp.mean(c * c, axis=-1, keepdims=True)
    return c * lax.rsqrt(var + 1e-5) * g + b


def _dense_body(x_ref, z0_ref, z1_ref, z2_ref, z3_ref, ea_ref,
                Wc_ref, W1_ref, W2_ref,
                b0_ref, b1_ref, b2_ref, g1_ref, be1_ref, g2_ref, be2_ref,
                o_ref):
    x = x_ref[...]
    h = jnp.concatenate(
        [x, z0_ref[...], z1_ref[...], z2_ref[...], z3_ref[...], ea_ref[...]],
        axis=1)
    acc = jnp.dot(h, Wc_ref[...], preferred_element_type=jnp.float32)
    acc = acc + b0_ref[...]
    out = _gelu(acc)
    h1 = _lnorm(x + out, g1_ref[...], be1_ref[...])
    f = _gelu(jnp.dot(h1, W1_ref[...], preferred_element_type=jnp.float32)
              + b1_ref[...])
    f = jnp.dot(f, W2_ref[...], preferred_element_type=jnp.float32) + b2_ref[...]
    o_ref[...] = _lnorm(h1 + f, g2_ref[...], be2_ref[...])


def _dense_core(x, z0, z1, z2, z3, ea, Wc, W1, W2, b0, b1, b2,
                g1, be1, g2, be2, blk):
    R = x.shape[0]
    Rp = ((R + blk - 1) // blk) * blk
    if Rp != R:
        pad = ((0, Rp - R), (0, 0))
        x = jnp.pad(x, pad)
        z0 = jnp.pad(z0, pad)
        z1 = jnp.pad(z1, pad)
        z2 = jnp.pad(z2, pad)
        z3 = jnp.pad(z3, pad)
        ea = jnp.pad(ea, pad)
    row_spec = pl.BlockSpec((blk, _D), lambda i: (i, 0))
    full = lambda shape: pl.BlockSpec(shape, lambda i: (0,) * len(shape))
    out = pl.pallas_call(
        _dense_body,
        grid=(Rp // blk,),
        in_specs=[
            row_spec, row_spec, row_spec, row_spec, row_spec, row_spec,
            full((6 * _D, _D)), full((_D, 4 * _D)), full((4 * _D, _D)),
            full((1, _D)), full((1, 4 * _D)), full((1, _D)),
            full((1, _D)), full((1, _D)), full((1, _D)), full((1, _D)),
        ],
        out_specs=row_spec,
        out_shape=jax.ShapeDtypeStruct((Rp, _D), jnp.float32),
    )(x, z0, z1, z2, z3, ea, Wc, W1, W2, b0, b1, b2, g1, be1, g2, be2)
    return out[:R] if Rp != R else out


def _half(xx, ea, src, dst, n, Ws, bs, Wk, bk, Wf, bf,
          W1, b1, W2, b2, g1, be1, g2, be2, blk):
    zs = []
    z = xx
    for _ in range(_KH):
        z = jax.ops.segment_sum(z[src], dst, num_segments=n)
        zs.append(z)
    Wc = jnp.concatenate([Ws, Wk[0], Wk[1], Wk[2], Wk[3], Wf], axis=0)
    b0 = (bs + bk.sum(axis=0) + bf)[None, :]
    return _dense_core(xx, zs[0], zs[1], zs[2], zs[3], ea, Wc, W1, W2,
                       b0, b1[None, :], b2[None, :],
                       g1[None, :], be1[None, :], g2[None, :], be2[None, :],
                       blk)


def kernel(x, lg_x, edge_index, lg_edge_index,
           node_Ws, node_bs, node_Wk, node_bk, node_Wf, node_bf,
           node_W1, node_b1, node_W2, node_b2, node_g1, node_be1,
           node_g2, node_be2,
           edge_Ws, edge_bs, edge_Wk, edge_bk, edge_Wf, edge_bf,
           edge_W1, edge_b1, edge_W2, edge_b2, edge_g1, edge_be1,
           edge_g2, edge_be2):
    n = x.shape[0]
    e = lg_x.shape[0]
    src = edge_index[0]
    dst = edge_index[1]
    lsrc = lg_edge_index[0]
    ldst = lg_edge_index[1]

    acc_rows = ((n + 1023) // 1024) * 1024
    eids = jnp.arange(e, dtype=jnp.int32)
    gi_inc, dv_inc = _pad_edges(jnp.concatenate([eids, eids]),
                                jnp.concatenate([src, dst]), acc_rows)
    node_edge_agg = _merge(_seg_partial(lg_x, gi_inc, dv_inc, acc_rows))[:n]
    gi_hop, dv_hop = _pad_edges(src, dst, acc_rows)
    z = jnp.pad(x, ((0, acc_rows - n), (0, 0)))
    zs_node = []
    for _ in range(_KH):
        z = _merge(_seg_partial(z, gi_hop, dv_hop, acc_rows))
        zs_node.append(z[:n])

    edge_node_agg = x[src] + x[dst]
    node_Wc = jnp.concatenate(
        [node_Ws, node_Wk[0], node_Wk[1], node_Wk[2], node_Wk[3], node_Wf],
        axis=0)
    node_b0 = (node_bs + node_bk.sum(axis=0) + node_bf)[None, :]
    next_x = _dense_core(x, zs_node[0], zs_node[1], zs_node[2], zs_node[3],
                         node_edge_agg, node_Wc, node_W1, node_W2,
                         node_b0, node_b1[None, :], node_b2[None, :],
                         node_g1[None, :], node_be1[None, :],
                         node_g2[None, :], node_be2[None, :], blk=512)
    next_lg_x = _half(lg_x, edge_node_agg, lsrc, ldst, lg_x.shape[0],
                      edge_Ws, edge_bs, edge_Wk, edge_bk, edge_Wf, edge_bf,
                      edge_W1, edge_b1, edge_W2, edge_b2,
                      edge_g1, edge_be1, edge_g2, edge_be2, blk=512)
    return (next_x, next_lg_x)


# speedup vs baseline: 1.4920x; 1.0717x over previous
import functools

import jax
import jax.numpy as jnp
from jax import lax
from jax.experimental import pallas as pl
from jax.experimental.pallas import tpu as pltpu
from jax.experimental.pallas import tpu_sc as plsc

_D = 128
_KH = 4
_C = 0.7978845608028654
_NSC = 2
_NT = 16
_B = 128


def _seg_partial(table, gidx, dstv, acc_rows):
    m = gidx.shape[0]
    assert m % (_NSC * _NT * _B) == 0
    assert acc_rows % (_NT * 8) == 0
    mt = m // (_NSC * _NT)
    nb = mt // _B
    rpt = acc_rows // _NT
    zeros = jnp.zeros((acc_rows, _D), jnp.float32)
    mesh = plsc.VectorSubcoreMesh(core_axis_name="c", subcore_axis_name="s")

    def body(table_r, gidx_r, dst_r, zero_r, out_r, gi_v, di_v, rows_v,
             acc, sem):
        c = lax.axis_index("c")
        t = lax.axis_index("s")
        r0 = t * rpt
        pltpu.sync_copy(zero_r.at[pl.ds(r0, rpt)], acc.at[pl.ds(r0, rpt)])
        plsc.subcore_barrier()
        base = c * (m // _NSC) + t * mt

        def step(j, carry):
            g0 = base + j * _B
            pltpu.sync_copy(gidx_r.at[pl.ds(g0, _B)], gi_v)
            pltpu.sync_copy(dst_r.at[pl.ds(g0, _B)], di_v)
            pltpu.async_copy(table_r.at[gi_v], rows_v, sem).wait()
            pltpu.sync_copy(rows_v, acc.at[di_v], add=True)
            return carry

        lax.fori_loop(0, nb, step, 0)
        plsc.subcore_barrier()
        pltpu.sync_copy(acc.at[pl.ds(r0, rpt)],
                        out_r.at[c].at[pl.ds(r0, rpt)])

    fn = pl.kernel(
        body,
        out_type=jax.ShapeDtypeStruct((_NSC, acc_rows, _D), jnp.float32),
        mesh=mesh,
        scratch_types=[
            pltpu.VMEM((_B,), jnp.int32),
            pltpu.VMEM((_B,), jnp.int32),
            pltpu.VMEM((_B, _D), jnp.float32),
            pltpu.VMEM_SHARED((acc_rows + 8, _D), jnp.float32),
            pltpu.SemaphoreType.DMA,
        ],
    )
    return fn(table, gidx, dstv, zeros)


_CH = 10240


def _compact(lsrc, ldst, nbk, cap):
    m = lsrc.shape[0]
    chunk = m // _NT
    nv = chunk // 16
    nbpc = nbk // _NSC
    mesh = plsc.VectorSubcoreMesh(core_axis_name="c", subcore_axis_name="s")

    def body(lsrc_r, ldst_r, glist_r, dlist_r, meta_r,
             ls_v, ld_v, gbat, dbat, mrow):
        c = lax.axis_index("c")
        t = lax.axis_index("s")
        lanes = lax.iota(jnp.int32, 16)
        pltpu.sync_copy(lsrc_r.at[pl.ds(t * chunk, chunk)], ls_v)
        pltpu.sync_copy(ldst_r.at[pl.ds(t * chunk, chunk)], ld_v)
        for bi in range(nbpc):
            b = bi * _NSC + c
            gbase = ((c * _NT + t) * nbpc + bi) * cap
            for k in range(10):
                gbat[pl.ds(k * 16, 16)] = lanes % 8
                dbat[pl.ds(k * 16, 16)] = jnp.full((16,), _CH, jnp.int32)

            def step(j, carry):
                fill, nbo = carry
                dv = ld_v[pl.ds(j * 16, 16)]
                sv = ls_v[pl.ds(j * 16, 16)]
                bb = dv // _CH
                msk = bb == b
                dloc = dv - b * _CH
                mi = msk.astype(jnp.int32)
                cs = plsc.cumsum(mi)
                pos = fill + cs - mi
                plsc.store_scatter(gbat, [pos], sv, mask=msk)
                plsc.store_scatter(dbat, [pos], dloc, mask=msk)
                fill = fill + cs[15]

                def fire(args):
                    fill, nbo = args
                    pltpu.sync_copy(
                        gbat.at[pl.ds(0, _B)],
                        glist_r.at[pl.ds(gbase + nbo * _B, _B)])
                    pltpu.sync_copy(
                        dbat.at[pl.ds(0, _B)],
                        dlist_r.at[pl.ds(gbase + nbo * _B, _B)])
                    rem = fill - _B
                    rv = gbat[pl.ds(_B, 16)]
                    rd = dbat[pl.ds(_B, 16)]
                    for k in range(10):
                        gbat[pl.ds(k * 16, 16)] = lanes % 8
                        dbat[pl.ds(k * 16, 16)] = jnp.full((16,), _CH,
                                                           jnp.int32)
                    pmask = lanes < rem
                    plsc.store_scatter(gbat, [lanes], rv, mask=pmask)
                    plsc.store_scatter(dbat, [lanes], rd, mask=pmask)
                    return rem, nbo + 1

                fill, nbo = lax.cond(fill >= _B, fire, lambda a: a,
                                     (fill, nbo))
                return fill, nbo

            fill, nbo = lax.fori_loop(0, nv, step,
                                      (jnp.int32(0), jnp.int32(0)))

            def tail(args):
                fill, nbo = args
                pltpu.sync_copy(
                    gbat.at[pl.ds(0, _B)],
                    glist_r.at[pl.ds(gbase + nbo * _B, _B)])
                pltpu.sync_copy(
                    dbat.at[pl.ds(0, _B)],
                    dlist_r.at[pl.ds(gbase + nbo * _B, _B)])
                return fill, nbo + 1

            fill, nbo = lax.cond(fill > 0, tail, lambda a: a, (fill, nbo))
            mv = mrow[...]
            mrow[...] = jnp.where(lanes == bi, nbo, mv)
        pltpu.sync_copy(mrow, meta_r.at[c * _NT + t])

    fn = pl.kernel(
        body,
        out_type=(
            jax.ShapeDtypeStruct((_NSC * _NT * nbpc * cap,), jnp.int32),
            jax.ShapeDtypeStruct((_NSC * _NT * nbpc * cap,), jnp.int32),
            jax.ShapeDtypeStruct((_NSC * _NT, 16), jnp.int32),
        ),
        mesh=mesh,
        compiler_params=pltpu.CompilerParams(needs_layout_passes=False),
        scratch_types=[
            pltpu.VMEM((chunk,), jnp.int32),
            pltpu.VMEM((chunk,), jnp.int32),
            pltpu.VMEM((160,), jnp.int32),
            pltpu.VMEM((160,), jnp.int32),
            pltpu.VMEM((16,), jnp.int32),
        ],
    )
    return fn(lsrc, ldst)


def _bucket_hop(table, glist, dlist, meta, nbk, cap):
    nbpc = nbk // _NSC
    rpt = _CH // _NT
    zeros = jnp.zeros((_CH, _D), jnp.float32)
    mesh = plsc.VectorSubcoreMesh(core_axis_name="c", subcore_axis_name="s")

    def body(table_r, glist_r, dlist_r, meta_r, zero_r, out_r,
             gi_v, di_v, rows_v, mrow, acc, sem):
        c = lax.axis_index("c")
        t = lax.axis_index("s")
        lanes = lax.iota(jnp.int32, 16)
        pltpu.sync_copy(meta_r.at[c * _NT + t], mrow)
        mv = mrow[...]
        r0 = t * rpt
        for bi in range(nbpc):
            b = bi * _NSC + c
            gbase = ((c * _NT + t) * nbpc + bi) * cap
            trips = mv[bi]
            pltpu.sync_copy(zero_r.at[pl.ds(r0, rpt)], acc.at[pl.ds(r0, rpt)])
            plsc.subcore_barrier()

            def step(j, carry):
                g0 = gbase + j * _B
                pltpu.sync_copy(glist_r.at[pl.ds(g0, _B)], gi_v)
                pltpu.sync_copy(dlist_r.at[pl.ds(g0, _B)], di_v)
                pltpu.async_copy(table_r.at[gi_v], rows_v, sem).wait()
                pltpu.sync_copy(rows_v, acc.at[di_v], add=True)
                return carry

            lax.fori_loop(0, trips, step, 0)
            plsc.subcore_barrier()
            pltpu.sync_copy(acc.at[pl.ds(r0, rpt)],
                            out_r.at[pl.ds(b * _CH + r0, rpt)])

    fn = pl.kernel(
        body,
        out_type=jax.ShapeDtypeStruct((nbk * _CH, _D), jnp.float32),
        mesh=mesh,
        compiler_params=pltpu.CompilerParams(needs_layout_passes=False),
        scratch_types=[
            pltpu.VMEM((_B,), jnp.int32),
            pltpu.VMEM((_B,), jnp.int32),
            pltpu.VMEM((_B, _D), jnp.float32),
            pltpu.VMEM((16,), jnp.int32),
            pltpu.VMEM_SHARED((_CH + 8, _D), jnp.float32),
            pltpu.SemaphoreType.DMA,
        ],
    )
    return fn(table, glist, dlist, meta, zeros)


def _merge_body(a_ref, b_ref, o_ref):
    o_ref[...] = a_ref[...] + b_ref[...]


def _merge(p):
    rows = p.shape[1]
    blk = 512
    spec = pl.BlockSpec((blk, _D), lambda i: (i, 0))
    return pl.pallas_call(
        _merge_body,
        grid=(rows // blk,),
        in_specs=[spec, spec],
        out_specs=spec,
        out_shape=jax.ShapeDtypeStruct((rows, _D), jnp.float32),
    )(p[0], p[1])


def _pad_edges(gidx, dstv, trash_row):
    m = gidx.shape[0]
    unit = _NSC * _NT * _B
    mp = ((m + unit - 1) // unit) * unit
    if mp != m:
        npad = mp - m
        gidx = jnp.concatenate([gidx, jnp.arange(npad, dtype=jnp.int32) % 8])
        dstv = jnp.concatenate(
            [dstv, jnp.full((npad,), trash_row, jnp.int32)])
    return gidx, dstv


def _gelu(v):
    return 0.5 * v * (1.0 + jnp.tanh(_C * (v + 0.044715 * v * v * v)))


def _lnorm(v, g, b):
    m = jnp.mean(v, axis=-1, keepdims=True)
    c = v - m
    var = jnp.mean(c * c, axis=-1, keepdims=True)
    return c * lax.rsqrt(var + 1e-5) * g + b


def _dense_body(x_ref, z0_ref, z1_ref, z2_ref, z3_ref, ea_ref,
                Wc_ref, W1_ref, W2_ref,
                b0_ref, b1_ref, b2_ref, g1_ref, be1_ref, g2_ref, be2_ref,
                o_ref):
    x = x_ref[...]
    h = jnp.concatenate(
        [x, z0_ref[...], z1_ref[...], z2_ref[...], z3_ref[...], ea_ref[...]],
        axis=1)
    acc = jnp.dot(h, Wc_ref[...], preferred_element_type=jnp.float32)
    acc = acc + b0_ref[...]
    out = _gelu(acc)
    h1 = _lnorm(x + out, g1_ref[...], be1_ref[...])
    f = _gelu(jnp.dot(h1, W1_ref[...], preferred_element_type=jnp.float32)
              + b1_ref[...])
    f = jnp.dot(f, W2_ref[...], preferred_element_type=jnp.float32) + b2_ref[...]
    o_ref[...] = _lnorm(h1 + f, g2_ref[...], be2_ref[...])


def _dense_core(x, z0, z1, z2, z3, ea, Wc, W1, W2, b0, b1, b2,
                g1, be1, g2, be2, blk):
    R = x.shape[0]
    Rp = ((R + blk - 1) // blk) * blk
    if Rp != R:
        pad = ((0, Rp - R), (0, 0))
        x = jnp.pad(x, pad)
        z0 = jnp.pad(z0, pad)
        z1 = jnp.pad(z1, pad)
        z2 = jnp.pad(z2, pad)
        z3 = jnp.pad(z3, pad)
        ea = jnp.pad(ea, pad)
    row_spec = pl.BlockSpec((blk, _D), lambda i: (i, 0))
    full = lambda shape: pl.BlockSpec(shape, lambda i: (0,) * len(shape))
    out = pl.pallas_call(
        _dense_body,
        grid=(Rp // blk,),
        in_specs=[
            row_spec, row_spec, row_spec, row_spec, row_spec, row_spec,
            full((6 * _D, _D)), full((_D, 4 * _D)), full((4 * _D, _D)),
            full((1, _D)), full((1, 4 * _D)), full((1, _D)),
            full((1, _D)), full((1, _D)), full((1, _D)), full((1, _D)),
        ],
        out_specs=row_spec,
        out_shape=jax.ShapeDtypeStruct((Rp, _D), jnp.float32),
    )(x, z0, z1, z2, z3, ea, Wc, W1, W2, b0, b1, b2, g1, be1, g2, be2)
    return out[:R] if Rp != R else out


def kernel(x, lg_x, edge_index, lg_edge_index,
           node_Ws, node_bs, node_Wk, node_bk, node_Wf, node_bf,
           node_W1, node_b1, node_W2, node_b2, node_g1, node_be1,
           node_g2, node_be2,
           edge_Ws, edge_bs, edge_Wk, edge_bk, edge_Wf, edge_bf,
           edge_W1, edge_b1, edge_W2, edge_b2, edge_g1, edge_be1,
           edge_g2, edge_be2):
    n = x.shape[0]
    e = lg_x.shape[0]
    src = edge_index[0]
    dst = edge_index[1]
    lsrc = lg_edge_index[0]
    ldst = lg_edge_index[1]

    acc_rows = ((n + 1023) // 1024) * 1024
    eids = jnp.arange(e, dtype=jnp.int32)
    gi_inc, dv_inc = _pad_edges(jnp.concatenate([eids, eids]),
                                jnp.concatenate([src, dst]), acc_rows)
    node_edge_agg = _merge(_seg_partial(lg_x, gi_inc, dv_inc, acc_rows))[:n]
    gi_hop, dv_hop = _pad_edges(src, dst, acc_rows)
    z = jnp.pad(x, ((0, acc_rows - n), (0, 0)))
    zs_node = []
    for _ in range(_KH):
        z = _merge(_seg_partial(z, gi_hop, dv_hop, acc_rows))
        zs_node.append(z[:n])

    node_Wc = jnp.concatenate(
        [node_Ws, node_Wk[0], node_Wk[1], node_Wk[2], node_Wk[3], node_Wf],
        axis=0)
    node_b0 = (node_bs + node_bk.sum(axis=0) + node_bf)[None, :]
    next_x = _dense_core(x, zs_node[0], zs_node[1], zs_node[2], zs_node[3],
                         node_edge_agg, node_Wc, node_W1, node_W2,
                         node_b0, node_b1[None, :], node_b2[None, :],
                         node_g1[None, :], node_be1[None, :],
                         node_g2[None, :], node_be2[None, :], blk=512)
    nbk = 32
    cap = 40960
    glist, dlist, meta = _compact(lsrc, ldst, nbk, cap)
    ze = lg_x
    zs_edge = []
    for _ in range(_KH):
        ze = _bucket_hop(ze, glist, dlist, meta, nbk, cap)
        zs_edge.append(ze[:e])

    rows_total = nbk * _CH
    idx = jnp.arange(rows_total, dtype=jnp.int32)
    valid = idx < e
    loc = jnp.where(valid, idx % _CH, _CH)
    sbp = jnp.where(valid, jnp.pad(src, (0, rows_total - e)), idx % 8)
    dbp = jnp.where(valid, jnp.pad(dst, (0, rows_total - e)), idx % 8)
    per_tile = 2 * _CH // _NT
    g_all = jnp.concatenate([sbp.reshape(nbk, _CH), dbp.reshape(nbk, _CH)],
                            axis=1).reshape(nbk, _NT, per_tile)
    l_all = jnp.concatenate([loc.reshape(nbk, _CH)] * 2,
                            axis=1).reshape(nbk, _NT, per_tile)
    g_inc = g_all.reshape(nbk // _NSC, _NSC, _NT, per_tile).transpose(
        1, 2, 0, 3).reshape(-1)
    l_inc = l_all.reshape(nbk // _NSC, _NSC, _NT, per_tile).transpose(
        1, 2, 0, 3).reshape(-1)
    meta_inc = jnp.full((_NSC * _NT, 16), per_tile // _B, jnp.int32)
    ea = _bucket_hop(x, g_inc, l_inc, meta_inc, nbk, per_tile)[:e]

    edge_Wc = jnp.concatenate(
        [edge_Ws, edge_Wk[0], edge_Wk[1], edge_Wk[2], edge_Wk[3], edge_Wf],
        axis=0)
    edge_b0 = (edge_bs + edge_bk.sum(axis=0) + edge_bf)[None, :]
    next_lg_x = _dense_core(lg_x, zs_edge[0], zs_edge[1], zs_edge[2],
                            zs_edge[3], ea, edge_Wc, edge_W1, edge_W2,
                            edge_b0, edge_b1[None, :], edge_b2[None, :],
                            edge_g1[None, :], edge_be1[None, :],
                            edge_g2[None, :], edge_be2[None, :], blk=512)
    return (next_x, next_lg_x)
